# baseline (speedup 1.0000x reference)
import functools

import jax
import jax.numpy as jnp
from jax import lax
from jax.experimental import pallas as pl
from jax.experimental.pallas import tpu as pltpu
from jax.experimental.pallas import tpu_sc as plsc


@functools.cache
def _make_gather(V, Ep, T, Bb, C):
    info = plsc.get_sparse_core_info()
    num_workers = info.num_cores * info.num_subcores
    per_w = T * Bb // num_workers
    n_chunks = per_w // C
    assert per_w % C == 0 and Bb % per_w == 0
    wpt = Bb // per_w

    mesh = plsc.VectorSubcoreMesh(core_axis_name="c", subcore_axis_name="s")

    @functools.partial(
        pl.kernel,
        mesh=mesh,
        out_type=jax.ShapeDtypeStruct((T, Bb, Ep), jnp.float32),
        scratch_types=[
            pltpu.VMEM((per_w,), jnp.int32),
            pltpu.VMEM((2, C, Ep), jnp.float32),
            pltpu.SemaphoreType.DMA((2,)),
            pltpu.SemaphoreType.DMA((2,)),
        ],
    )
    def gather_kernel(table_hbm, idx_hbm, out_hbm, idx_v, rows_v, gsem, wsem):
        wid = lax.axis_index("s") * info.num_cores + lax.axis_index("c")
        base = wid * per_w
        t_id = wid // wpt
        b_base = (wid % wpt) * per_w
        pltpu.sync_copy(idx_hbm.at[pl.ds(base, per_w)], idx_v)

        def body(i, carry):
            s = lax.rem(i, 2)

            @pl.when(i >= 2)
            def _():
                pltpu.make_async_copy(
                    rows_v.at[s],
                    out_hbm.at[t_id, pl.ds(b_base + (i - 2) * C, C), :],
                    wsem.at[s],
                ).wait()

            pltpu.async_copy(
                table_hbm.at[idx_v.at[pl.ds(i * C, C)]], rows_v.at[s], gsem.at[s]
            ).wait()
            pltpu.make_async_copy(
                rows_v.at[s],
                out_hbm.at[t_id, pl.ds(b_base + i * C, C), :],
                wsem.at[s],
            ).start()
            return carry

        lax.fori_loop(0, n_chunks, body, 0)
        for i in range(max(n_chunks - 2, 0), n_chunks):
            s = i % 2
            pltpu.make_async_copy(
                rows_v.at[s],
                out_hbm.at[t_id, pl.ds(b_base + i * C, C), :],
                wsem.at[s],
            ).wait()

    return gather_kernel


def _head_body(emb_ref, wt_ref, b_ref, out_ref):
    _, vc, bb = out_ref.shape
    acc = (
        lax.dot_general(
            wt_ref[...],
            emb_ref[0],
            dimension_numbers=(((1,), (1,)), ((), ())),
            preferred_element_type=jnp.float32,
        )
        + b_ref[...]
    )
    out_ref[...] = acc.reshape(1, vc, bb)


def _head_body_alias(emb_ref, wt_ref, b_ref, prev_ref, out_ref):
    _head_body(emb_ref, wt_ref, b_ref, out_ref)


@functools.cache
def _make_head_t(T, Tg, s0, Ep, Bb, Vo, VC, BC, alias):
    grid = (Tg, Bb // BC, Vo // VC)
    in_specs = [
        pl.BlockSpec((1, BC, Ep), lambda t, k, j: (t, k, 0)),
        pl.BlockSpec((VC, Ep), lambda t, k, j: (j, 0)),
        pl.BlockSpec((VC, 1), lambda t, k, j: (j, 0)),
    ]
    kwargs = {}
    if alias:
        in_specs.append(pl.BlockSpec(memory_space=pl.ANY))
        kwargs["input_output_aliases"] = {3: 0}
    return pl.pallas_call(
        _head_body_alias if alias else _head_body,
        grid=grid,
        in_specs=in_specs,
        out_specs=pl.BlockSpec((1, VC, BC), lambda t, k, j: (t + s0, j, k)),
        out_shape=jax.ShapeDtypeStruct((T, Vo, Bb), jnp.float32),
        **kwargs,
    )


def kernel(idx, tok_table, pos_table, W, b):
    Bb, T = idx.shape
    V, E = tok_table.shape
    Vo = W.shape[1]
    Ep = 128
    NS = 2
    Tg = T // NS

    tok_p = jnp.pad(tok_table, ((0, 0), (0, Ep - E)))
    Wt_p = jnp.pad(W.T, ((0, 0), (0, Ep - E)))
    bcol = b.reshape(Vo, 1)
    idx_t = idx.T.reshape(-1).astype(jnp.int32)

    gather = _make_gather(V, Ep, Tg, Bb, 256)
    embs = [
        gather(tok_p, lax.slice(idx_t, (s * Tg * Bb,), ((s + 1) * Tg * Bb,)))
        for s in range(NS)
    ]
    out_tr = _make_head_t(T, Tg, 0, Ep, Bb, Vo, 200, 8192, False)(
        embs[0], Wt_p, bcol
    )
    for s in range(1, NS):
        out_tr = _make_head_t(T, Tg, s * Tg, Ep, Bb, Vo, 200, 8192, True)(
            embs[s], Wt_p, bcol, out_tr
        )
    return jnp.transpose(out_tr, (2, 0, 1))


# speedup vs baseline: 1.0159x; 1.0159x over previous
import functools

import jax
import jax.numpy as jnp
from jax import lax
from jax.experimental import pallas as pl
from jax.experimental.pallas import tpu as pltpu
from jax.experimental.pallas import tpu_sc as plsc


@functools.cache
def _make_gather(V, Ep, T, Bb, C):
    info = plsc.get_sparse_core_info()
    num_workers = info.num_cores * info.num_subcores
    per_w = T * Bb // num_workers
    n_chunks = per_w // C
    assert per_w % C == 0 and Bb % per_w == 0
    wpt = Bb // per_w

    mesh = plsc.VectorSubcoreMesh(core_axis_name="c", subcore_axis_name="s")

    @functools.partial(
        pl.kernel,
        mesh=mesh,
        out_type=jax.ShapeDtypeStruct((T, Bb, Ep), jnp.float32),
        scratch_types=[
            pltpu.VMEM((per_w,), jnp.int32),
            pltpu.VMEM((2, C, Ep), jnp.float32),
            pltpu.SemaphoreType.DMA((2,)),
            pltpu.SemaphoreType.DMA((2,)),
        ],
    )
    def gather_kernel(table_hbm, idx_hbm, out_hbm, idx_v, rows_v, gsem, wsem):
        wid = lax.axis_index("s") * info.num_cores + lax.axis_index("c")
        base = wid * per_w
        t_id = wid // wpt
        b_base = (wid % wpt) * per_w
        pltpu.sync_copy(idx_hbm.at[pl.ds(base, per_w)], idx_v)

        def body(i, carry):
            s = lax.rem(i, 2)

            @pl.when(i >= 2)
            def _():
                pltpu.make_async_copy(
                    rows_v.at[s],
                    out_hbm.at[t_id, pl.ds(b_base + (i - 2) * C, C), :],
                    wsem.at[s],
                ).wait()

            pltpu.async_copy(
                table_hbm.at[idx_v.at[pl.ds(i * C, C)]], rows_v.at[s], gsem.at[s]
            ).wait()
            pltpu.make_async_copy(
                rows_v.at[s],
                out_hbm.at[t_id, pl.ds(b_base + i * C, C), :],
                wsem.at[s],
            ).start()
            return carry

        lax.fori_loop(0, n_chunks, body, 0)
        for i in range(max(n_chunks - 2, 0), n_chunks):
            s = i % 2
            pltpu.make_async_copy(
                rows_v.at[s],
                out_hbm.at[t_id, pl.ds(b_base + i * C, C), :],
                wsem.at[s],
            ).wait()

    return gather_kernel


def _head_body(emb_ref, wt_ref, b_ref, out_ref):
    _, vc, bb = out_ref.shape
    acc = (
        lax.dot_general(
            wt_ref[...],
            emb_ref[0],
            dimension_numbers=(((1,), (1,)), ((), ())),
            preferred_element_type=jnp.float32,
        )
        + b_ref[...]
    )
    out_ref[...] = acc.reshape(1, vc, bb)


def _head_body_alias(emb_ref, wt_ref, b_ref, prev_ref, out_ref):
    _head_body(emb_ref, wt_ref, b_ref, out_ref)


@functools.cache
def _make_head_t(T, Tg, s0, Ep, Bb, Vo, VC, BC, alias):
    grid = (Tg, Bb // BC, Vo // VC)
    in_specs = [
        pl.BlockSpec((1, BC, Ep), lambda t, k, j: (t, k, 0)),
        pl.BlockSpec((VC, Ep), lambda t, k, j: (j, 0)),
        pl.BlockSpec((VC, 1), lambda t, k, j: (j, 0)),
    ]
    kwargs = {}
    if alias:
        in_specs.append(pl.BlockSpec(memory_space=pl.ANY))
        kwargs["input_output_aliases"] = {3: 0}
    return pl.pallas_call(
        _head_body_alias if alias else _head_body,
        grid=grid,
        in_specs=in_specs,
        out_specs=pl.BlockSpec((1, VC, BC), lambda t, k, j: (t + s0, j, k)),
        out_shape=jax.ShapeDtypeStruct((T, Vo, Bb), jnp.float32),
        **kwargs,
    )


def kernel(idx, tok_table, pos_table, W, b):
    Bb, T = idx.shape
    V, E = tok_table.shape
    Vo = W.shape[1]
    Ep = 128
    splits = (1, 1, 2, 4)
    assert sum(splits) == T

    tok_p = jnp.pad(tok_table, ((0, 0), (0, Ep - E)))
    Wt_p = jnp.pad(W.T, ((0, 0), (0, Ep - E)))
    bcol = b.reshape(Vo, 1)
    idx_t = idx.T.reshape(-1).astype(jnp.int32)

    embs, s0 = [], 0
    for tg in splits:
        embs.append(
            _make_gather(V, Ep, tg, Bb, 256)(
                tok_p, lax.slice(idx_t, (s0 * Bb,), ((s0 + tg) * Bb,))
            )
        )
        s0 += tg
    out_tr, s0 = None, 0
    for i, tg in enumerate(splits):
        if i == 0:
            out_tr = _make_head_t(T, tg, 0, Ep, Bb, Vo, 200, 8192, False)(
                embs[0], Wt_p, bcol
            )
        else:
            out_tr = _make_head_t(T, tg, s0, Ep, Bb, Vo, 200, 8192, True)(
                embs[i], Wt_p, bcol, out_tr
            )
        s0 += tg
    return jnp.transpose(out_tr, (2, 0, 1))


# speedup vs baseline: 1.0164x; 1.0005x over previous
import functools

import jax
import jax.numpy as jnp
from jax import lax
from jax.experimental import pallas as pl
from jax.experimental.pallas import tpu as pltpu
from jax.experimental.pallas import tpu_sc as plsc


@functools.cache
def _make_gather(V, E, Ep, T, Bb, C):
    info = plsc.get_sparse_core_info()
    L = info.num_lanes
    num_workers = info.num_cores * info.num_subcores
    per_w = T * Bb // num_workers
    n_chunks = per_w // C
    assert per_w % C == 0 and Bb % per_w == 0
    assert E % L == 0
    wpt = Bb // per_w

    mesh = plsc.VectorSubcoreMesh(core_axis_name="c", subcore_axis_name="s")

    @functools.partial(
        pl.kernel,
        mesh=mesh,
        out_type=jax.ShapeDtypeStruct((T, Bb, E), jnp.float32),
        scratch_types=[
            pltpu.VMEM((per_w,), jnp.int32),
            pltpu.VMEM((2, C, Ep), jnp.float32),
            pltpu.VMEM((2, C, E), jnp.float32),
            pltpu.SemaphoreType.DMA((2,)),
            pltpu.SemaphoreType.DMA((2,)),
        ],
    )
    def gather_kernel(
        table_hbm, idx_hbm, out_hbm, idx_v, rows_v, crows_v, gsem, wsem
    ):
        wid = lax.axis_index("s") * info.num_cores + lax.axis_index("c")
        base = wid * per_w
        t_id = wid // wpt
        b_base = (wid % wpt) * per_w
        pltpu.sync_copy(idx_hbm.at[pl.ds(base, per_w)], idx_v)

        def body(i, carry):
            s = lax.rem(i, 2)

            @pl.when(i >= 2)
            def _():
                pltpu.make_async_copy(
                    crows_v.at[s],
                    out_hbm.at[t_id, pl.ds(b_base + (i - 2) * C, C), :],
                    wsem.at[s],
                ).wait()

            pltpu.async_copy(
                table_hbm.at[idx_v.at[pl.ds(i * C, C)]], rows_v.at[s], gsem.at[s]
            ).wait()

            def compact(r, carry2):
                for e in range(E // L):
                    crows_v[s, r, pl.ds(e * L, L)] = rows_v[s, r, pl.ds(e * L, L)]
                return carry2

            lax.fori_loop(0, C, compact, 0)
            pltpu.make_async_copy(
                crows_v.at[s],
                out_hbm.at[t_id, pl.ds(b_base + i * C, C), :],
                wsem.at[s],
            ).start()
            return carry

        lax.fori_loop(0, n_chunks, body, 0)
        for i in range(max(n_chunks - 2, 0), n_chunks):
            s = i % 2
            pltpu.make_async_copy(
                crows_v.at[s],
                out_hbm.at[t_id, pl.ds(b_base + i * C, C), :],
                wsem.at[s],
            ).wait()

    return gather_kernel


def _head_body(emb_ref, wt_ref, b_ref, out_ref):
    _, vc, bb = out_ref.shape
    acc = (
        lax.dot_general(
            wt_ref[...],
            emb_ref[0],
            dimension_numbers=(((1,), (1,)), ((), ())),
            preferred_element_type=jnp.float32,
        )
        + b_ref[...]
    )
    out_ref[...] = acc.reshape(1, vc, bb)


def _head_body_alias(emb_ref, wt_ref, b_ref, prev_ref, out_ref):
    _head_body(emb_ref, wt_ref, b_ref, out_ref)


@functools.cache
def _make_head_t(T, Tg, s0, Ep, Bb, Vo, VC, BC, alias):
    grid = (Tg, Bb // BC, Vo // VC)
    in_specs = [
        pl.BlockSpec((1, BC, Ep), lambda t, k, j: (t, k, 0)),
        pl.BlockSpec((VC, Ep), lambda t, k, j: (j, 0)),
        pl.BlockSpec((VC, 1), lambda t, k, j: (j, 0)),
    ]
    kwargs = {}
    if alias:
        in_specs.append(pl.BlockSpec(memory_space=pl.ANY))
        kwargs["input_output_aliases"] = {3: 0}
    return pl.pallas_call(
        _head_body_alias if alias else _head_body,
        grid=grid,
        in_specs=in_specs,
        out_specs=pl.BlockSpec((1, VC, BC), lambda t, k, j: (t + s0, j, k)),
        out_shape=jax.ShapeDtypeStruct((T, Vo, Bb), jnp.float32),
        **kwargs,
    )


def kernel(idx, tok_table, pos_table, W, b):
    Bb, T = idx.shape
    V, E = tok_table.shape
    Vo = W.shape[1]
    Ep = 128
    splits = (1, 1, 2, 4)
    assert sum(splits) == T

    tok_p = jnp.pad(tok_table, ((0, 0), (0, Ep - E)))
    Wt = W.T
    bcol = b.reshape(Vo, 1)
    idx_t = idx.T.reshape(-1).astype(jnp.int32)

    embs, s0 = [], 0
    for tg in splits:
        embs.append(
            _make_gather(V, E, Ep, tg, Bb, 128)(
                tok_p, lax.slice(idx_t, (s0 * Bb,), ((s0 + tg) * Bb,))
            )
        )
        s0 += tg
    out_tr, s0 = None, 0
    for i, tg in enumerate(splits):
        if i == 0:
            out_tr = _make_head_t(T, tg, 0, E, Bb, Vo, 200, 8192, False)(
                embs[0], Wt, bcol
            )
        else:
            out_tr = _make_head_t(T, tg, s0, E, Bb, Vo, 200, 8192, True)(
                embs[i], Wt, bcol, out_tr
            )
        s0 += tg
    return jnp.transpose(out_tr, (2, 0, 1))
